# Initial kernel scaffold; baseline (speedup 1.0000x reference)
import functools

import jax
import jax.numpy as jnp
from jax import lax
from jax.experimental import pallas as pl
from jax.experimental.pallas import tpu as pltpu
from jax.experimental.pallas import tpu_sc as plsc

B, L1, L2, D = 1024, 50, 20, 64
V_MED, V_UNITS = 100000, 100
NC, NS = 2, 16
NW = NC * NS
B_PER_W = B // NW
EPB = L1 * L2
EPB_PAD = 1024
GCHUNK = 128


def _sc_body(ids_hbm, units_hbm, dose_hbm, med_tab_hbm, utab_hbm,
             out_hbm, pooled_hbm,
             idx_v, units_v, dose_v, rows_v, utab_v, oacc_v, pooled_v, sem):
    wid = lax.axis_index("s") * NC + lax.axis_index("c")
    col16 = lax.iota(jnp.int32, 16)
    zeros16 = jnp.zeros((16,), jnp.float32)

    pltpu.sync_copy(utab_hbm, utab_v)

    @pl.loop(0, B_PER_W)
    def _b_loop(lb):
        b = wid * B_PER_W + lb
        base = b * EPB
        pltpu.sync_copy(ids_hbm.at[pl.ds(base, EPB)], idx_v.at[pl.ds(0, EPB)])
        idx_v[pl.ds(EPB, 16)] = jnp.zeros((16,), jnp.int32)
        idx_v[pl.ds(EPB + 8, 16)] = jnp.zeros((16,), jnp.int32)
        pltpu.sync_copy(units_hbm.at[pl.ds(base, EPB)],
                        units_v.at[pl.ds(0, EPB)])
        pltpu.sync_copy(dose_hbm.at[pl.ds(base, EPB)],
                        dose_v.at[pl.ds(0, EPB)])

        descs = []
        for j in range(EPB_PAD // GCHUNK):
            descs.append(pltpu.async_copy(
                med_tab_hbm.at[idx_v.at[pl.ds(j * GCHUNK, GCHUNK)]],
                rows_v.at[pl.ds(j * GCHUNK, GCHUNK)],
                sem))
        for d_ in descs:
            d_.wait()

        def l1_body(l1, pooled):
            acc = [zeros16, zeros16, zeros16, zeros16]
            for l2 in range(L2):
                k = l1 * L2 + l2
                dsc = jnp.full((16,), dose_v[k], jnp.float32)
                ubase = jnp.full((16,), units_v[k] * D, jnp.int32) + col16
                for j in range(4):
                    m = rows_v[k, pl.ds(16 * j, 16)]
                    u = plsc.load_gather(utab_v, [ubase + (16 * j)])
                    acc[j] = acc[j] + m * u * dsc
            for j in range(4):
                oacc_v[pl.ds(l1 * D + 16 * j, 16)] = acc[j]
            return [pooled[j] + acc[j] for j in range(4)]

        pooled = lax.fori_loop(0, L1, l1_body,
                               [zeros16, zeros16, zeros16, zeros16])
        for j in range(4):
            pooled_v[pl.ds(16 * j, 16)] = pooled[j]
        pltpu.sync_copy(oacc_v, out_hbm.at[pl.ds(b * L1 * D, L1 * D)])
        pltpu.sync_copy(pooled_v, pooled_hbm.at[pl.ds(b * D, D)])


def _head_body(pooled_ref, w_ref, b_ref, out_ref):
    logits = jnp.dot(pooled_ref[...], w_ref[...].T,
                     preferred_element_type=jnp.float32) + b_ref[0, 0]
    out_ref[...] = jax.nn.sigmoid(logits)


def kernel(medication_ids, dose, units, med_table, units_table, W, b):
    ids_flat = medication_ids.astype(jnp.int32).reshape(B * L1 * L2)
    units_flat = units.astype(jnp.int32).reshape(B * L1 * L2)
    dose_flat = dose.reshape(B * L1 * L2)
    utab_flat = units_table.reshape((V_UNITS + 1) * D)

    sc = pl.kernel(
        _sc_body,
        out_type=[
            jax.ShapeDtypeStruct((B * L1 * D,), jnp.float32),
            jax.ShapeDtypeStruct((B * D,), jnp.float32),
        ],
        mesh=plsc.VectorSubcoreMesh(core_axis_name="c", subcore_axis_name="s"),
        scratch_types=[
            pltpu.VMEM((EPB_PAD,), jnp.int32),
            pltpu.VMEM((EPB_PAD,), jnp.int32),
            pltpu.VMEM((EPB_PAD,), jnp.float32),
            pltpu.VMEM((EPB_PAD, D), jnp.float32),
            pltpu.VMEM(((V_UNITS + 1) * D,), jnp.float32),
            pltpu.VMEM((L1 * D,), jnp.float32),
            pltpu.VMEM((D,), jnp.float32),
            pltpu.SemaphoreType.DMA,
        ],
    )
    out_flat, pooled_flat = sc(ids_flat, units_flat, dose_flat,
                               med_table, utab_flat)

    pooled = pooled_flat.reshape(B, D)
    outcome = pl.pallas_call(
        _head_body,
        out_shape=jax.ShapeDtypeStruct((B, 1), jnp.float32),
    )(pooled, W, b.reshape(1, 1))

    return out_flat.reshape(B, L1, D), outcome


# speedup vs baseline: 9.8747x; 9.8747x over previous
import functools

import jax
import jax.numpy as jnp
from jax import lax
from jax.experimental import pallas as pl
from jax.experimental.pallas import tpu as pltpu
from jax.experimental.pallas import tpu_sc as plsc

B, L1, L2, D = 1024, 50, 20, 64
V_MED, V_UNITS = 100000, 100
NC, NS = 2, 16
NW = NC * NS
B_PER_W = B // NW
EPB = L1 * L2
EPB_PAD = 1024
GCHUNK = 128
L2P = 32


def _sc_body(ids_hbm, units_hbm, dose_hbm, med_tab_hbm, utab_hbm,
             out_hbm, pooled_hbm,
             idx_v, units_v, dose_v, rows_v, utab_v, oacc_v, pooled_v, sem):
    wid = lax.axis_index("s") * NC + lax.axis_index("c")
    col16 = lax.iota(jnp.int32, 16)
    zeros16 = jnp.zeros((16,), jnp.float32)

    pltpu.sync_copy(utab_hbm, utab_v)

    @pl.loop(0, B_PER_W)
    def _b_loop(lb):
        b = wid * B_PER_W + lb
        base = b * EPB
        pltpu.sync_copy(ids_hbm.at[pl.ds(base, EPB)], idx_v.at[pl.ds(0, EPB)])
        idx_v[pl.ds(EPB, 16)] = jnp.zeros((16,), jnp.int32)
        idx_v[pl.ds(EPB + 8, 16)] = jnp.zeros((16,), jnp.int32)
        pbase = b * (L1 * L2P)
        pltpu.sync_copy(units_hbm.at[pl.ds(pbase, L1 * L2P)], units_v)
        pltpu.sync_copy(dose_hbm.at[pl.ds(pbase, L1 * L2P)], dose_v)

        descs = []
        for j in range(EPB_PAD // GCHUNK):
            descs.append(pltpu.async_copy(
                med_tab_hbm.at[idx_v.at[pl.ds(j * GCHUNK, GCHUNK)]],
                rows_v.at[pl.ds(j * GCHUNK, GCHUNK)],
                sem))
        for d_ in descs:
            d_.wait()

        def l1_body(l1, pooled):
            dv = [dose_v[pl.ds(l1 * L2P, 16)],
                  dose_v[pl.ds(l1 * L2P + 16, 16)]]
            uv = [units_v[pl.ds(l1 * L2P, 16)],
                  units_v[pl.ds(l1 * L2P + 16, 16)]]
            acc = [zeros16, zeros16, zeros16, zeros16]
            for l2 in range(L2):
                k = l1 * L2 + l2
                dsc = jnp.full((16,), dv[l2 // 16][l2 % 16], jnp.float32)
                ub = uv[l2 // 16][l2 % 16] * D
                for j in range(4):
                    m = rows_v[k, pl.ds(16 * j, 16)]
                    u = utab_v[pl.ds(ub + 16 * j, 16)]
                    acc[j] = acc[j] + m * u * dsc
            for j in range(4):
                oacc_v[pl.ds(l1 * D + 16 * j, 16)] = acc[j]
            return [pooled[j] + acc[j] for j in range(4)]

        pooled = lax.fori_loop(0, L1, l1_body,
                               [zeros16, zeros16, zeros16, zeros16])
        for j in range(4):
            pooled_v[pl.ds(16 * j, 16)] = pooled[j]
        pltpu.sync_copy(oacc_v, out_hbm.at[pl.ds(b * L1 * D, L1 * D)])
        pltpu.sync_copy(pooled_v, pooled_hbm.at[pl.ds(b * D, D)])


def _head_body(pooled_ref, w_ref, b_ref, out_ref):
    logits = jnp.dot(pooled_ref[...], w_ref[...].T,
                     preferred_element_type=jnp.float32) + b_ref[0, 0]
    out_ref[...] = jax.nn.sigmoid(logits)


_HEAD_PAD = 8


def kernel(medication_ids, dose, units, med_table, units_table, W, b):
    ids_flat = medication_ids.astype(jnp.int32).reshape(B * L1 * L2)
    pad = ((0, 0), (0, 0), (0, L2P - L2))
    units_flat = jnp.pad(units.astype(jnp.int32), pad).reshape(B * L1 * L2P)
    dose_flat = jnp.pad(dose, pad).reshape(B * L1 * L2P)
    utab_flat = units_table.reshape((V_UNITS + 1) * D)

    sc = pl.kernel(
        _sc_body,
        out_type=[
            jax.ShapeDtypeStruct((B * L1 * D,), jnp.float32),
            jax.ShapeDtypeStruct((B * D,), jnp.float32),
        ],
        mesh=plsc.VectorSubcoreMesh(core_axis_name="c", subcore_axis_name="s"),
        compiler_params=pltpu.CompilerParams(use_tc_tiling_on_sc=False),
        scratch_types=[
            pltpu.VMEM((EPB_PAD,), jnp.int32),
            pltpu.VMEM((L1 * L2P,), jnp.int32),
            pltpu.VMEM((L1 * L2P,), jnp.float32),
            pltpu.VMEM((EPB_PAD, D), jnp.float32),
            pltpu.VMEM(((V_UNITS + 1) * D,), jnp.float32),
            pltpu.VMEM((L1 * D,), jnp.float32),
            pltpu.VMEM((D,), jnp.float32),
            pltpu.SemaphoreType.DMA,
        ],
    )
    out_flat, pooled_flat = sc(ids_flat, units_flat, dose_flat,
                               med_table, utab_flat)

    pooled = pooled_flat.reshape(B, D)
    w_pad = jnp.pad(W, ((0, _HEAD_PAD - 1), (0, 0)))
    outcome = pl.pallas_call(
        _head_body,
        out_shape=jax.ShapeDtypeStruct((B, _HEAD_PAD), jnp.float32),
    )(pooled, w_pad, b.reshape(1, 1))

    return out_flat.reshape(B, L1, D), outcome[:, :1]


# speedup vs baseline: 14.3135x; 1.4495x over previous
import functools

import jax
import jax.numpy as jnp
from jax import lax
from jax.experimental import pallas as pl
from jax.experimental.pallas import tpu as pltpu
from jax.experimental.pallas import tpu_sc as plsc

B, L1, L2, D = 1024, 50, 20, 64
V_MED, V_UNITS = 100000, 100
NC, NS = 2, 16
NW = NC * NS
B_PER_W = B // NW
EPB = L1 * L2
EPB_PAD = 1024
GCHUNK = 128
NG = EPB_PAD // GCHUNK
L2P = 32


def _sc_body(ids_hbm, units_hbm, dose_hbm, med_tab_hbm, utab_hbm,
             out_hbm, pooled_hbm, *refs):
    (idx0, idx1, stag_u0, stag_u1, stag_d0, stag_d1,
     units0, units1, dose0, dose1, rows0, rows1, oacc0, oacc1,
     utab_v, pooled_v,
     sem_ids0, sem_ids1, sem_inp0, sem_inp1,
     sem_g0, sem_g1, sem_out0, sem_out1) = refs
    bufs = [
        dict(idx=idx0, stag_u=stag_u0, stag_d=stag_d0, units=units0,
             dose=dose0, rows=rows0, oacc=oacc0, sem_ids=sem_ids0,
             sem_inp=sem_inp0, sem_g=sem_g0, sem_out=sem_out0),
        dict(idx=idx1, stag_u=stag_u1, stag_d=stag_d1, units=units1,
             dose=dose1, rows=rows1, oacc=oacc1, sem_ids=sem_ids1,
             sem_inp=sem_inp1, sem_g=sem_g1, sem_out=sem_out1),
    ]
    wid = lax.axis_index("s") * NC + lax.axis_index("c")
    b0 = wid * B_PER_W
    col16 = lax.iota(jnp.int32, 16)
    col2 = col16 * 2
    zeros16 = jnp.zeros((16,), jnp.float32)
    himask = jnp.full((16,), -65536, jnp.int32)

    pltpu.sync_copy(utab_hbm, utab_v)
    for bf in bufs:
        bf["idx"][pl.ds(EPB, 16)] = jnp.zeros((16,), jnp.int32)
        bf["idx"][pl.ds(EPB + 8, 16)] = jnp.zeros((16,), jnp.int32)
    d_ids0 = pltpu.async_copy(ids_hbm.at[pl.ds(b0 * EPB, EPB)],
                              idx0.at[pl.ds(0, EPB)], sem_ids0)
    pltpu.async_copy(ids_hbm.at[pl.ds((b0 + 1) * EPB, EPB)],
                     idx1.at[pl.ds(0, EPB)], sem_ids1)
    d_ids0.wait()
    for j in range(NG):
        pltpu.async_copy(
            med_tab_hbm.at[idx0.at[pl.ds(j * GCHUNK, GCHUNK)]],
            rows0.at[pl.ds(j * GCHUNK, GCHUNK)], sem_g0)
    pltpu.async_copy(units_hbm.at[pl.ds(b0 * EPB, EPB)],
                     stag_u0.at[pl.ds(0, EPB)], sem_inp0)
    pltpu.async_copy(dose_hbm.at[pl.ds(b0 * EPB, EPB)],
                     stag_d0.at[pl.ds(0, EPB)], sem_inp0)
    pltpu.async_copy(oacc0, out_hbm.at[pl.ds(b0 * L1 * D, L1 * D)],
                     sem_out0)
    pltpu.async_copy(oacc1, out_hbm.at[pl.ds((b0 + 1) * L1 * D, L1 * D)],
                     sem_out1)

    def stage(lb, cur, nxt):
        b = b0 + lb
        @pl.when(lb + 1 < B_PER_W)
        def _prefetch():
            bn = b + 1
            pltpu.make_async_copy(
                ids_hbm.at[pl.ds(bn * EPB, EPB)],
                nxt["idx"].at[pl.ds(0, EPB)], nxt["sem_ids"]).wait()
            for j in range(NG):
                pltpu.async_copy(
                    med_tab_hbm.at[nxt["idx"].at[pl.ds(j * GCHUNK, GCHUNK)]],
                    nxt["rows"].at[pl.ds(j * GCHUNK, GCHUNK)], nxt["sem_g"])
            pltpu.async_copy(units_hbm.at[pl.ds(bn * EPB, EPB)],
                             nxt["stag_u"].at[pl.ds(0, EPB)], nxt["sem_inp"])
            pltpu.async_copy(dose_hbm.at[pl.ds(bn * EPB, EPB)],
                             nxt["stag_d"].at[pl.ds(0, EPB)], nxt["sem_inp"])

        pltpu.make_async_copy(units_hbm.at[pl.ds(b * EPB, EPB)],
                              cur["stag_u"].at[pl.ds(0, EPB)],
                              cur["sem_inp"]).wait()
        pltpu.make_async_copy(dose_hbm.at[pl.ds(b * EPB, EPB)],
                              cur["stag_d"].at[pl.ds(0, EPB)],
                              cur["sem_inp"]).wait()

        @pl.loop(0, L1 * L2P // 16)
        def _relayout(g):
            p = jnp.full((16,), g * 16, jnp.int32) + col16
            src = (p >> 5) * L2 + (p & (L2P - 1))
            cur["dose"][pl.ds(g * 16, 16)] = plsc.load_gather(
                cur["stag_d"], [src])
            cur["units"][pl.ds(g * 16, 16)] = plsc.load_gather(
                cur["stag_u"], [src])

        for j in range(NG):
            pltpu.make_async_copy(
                med_tab_hbm.at[cur["idx"].at[pl.ds(j * GCHUNK, GCHUNK)]],
                cur["rows"].at[pl.ds(j * GCHUNK, GCHUNK)], cur["sem_g"]).wait()
        pltpu.make_async_copy(
            cur["oacc"], out_hbm.at[pl.ds(b * L1 * D, L1 * D)],
            cur["sem_out"]).wait()

        def l1_body(l1, pooled):
            dv = [cur["dose"][pl.ds(l1 * L2P, 16)],
                  cur["dose"][pl.ds(l1 * L2P + 16, 16)]]
            uv = [cur["units"][pl.ds(l1 * L2P, 16)],
                  cur["units"][pl.ds(l1 * L2P + 16, 16)]]
            acc = [zeros16, zeros16, zeros16, zeros16]
            for l2 in range(L2):
                k = l1 * L2 + l2
                dvec = jnp.full((16,), dv[l2 // 16][l2 % 16], jnp.float32)
                ub = uv[l2 // 16][l2 % 16] * D
                for c in range(2):
                    v = plsc.bitcast(cur["rows"][k, pl.ds(32 * c, 32)],
                                     jnp.int32)
                    m_e = plsc.bitcast(lax.shift_left(v, 16), jnp.float32)
                    m_o = plsc.bitcast(v & himask, jnp.float32)
                    u_e = utab_v[pl.ds(ub + 32 * c, 16)]
                    u_o = utab_v[pl.ds(ub + 32 * c + 16, 16)]
                    acc[2 * c] = acc[2 * c] + m_e * u_e * dvec
                    acc[2 * c + 1] = acc[2 * c + 1] + m_o * u_o * dvec
            base = l1 * D
            for c in range(2):
                for par in range(2):
                    plsc.store_scatter(
                        cur["oacc"],
                        [jnp.full((16,), base + 32 * c + par, jnp.int32)
                         + col2],
                        acc[2 * c + par])
            return [pooled[i] + acc[i] for i in range(4)]

        pooled = lax.fori_loop(0, L1, l1_body,
                               [zeros16, zeros16, zeros16, zeros16])
        pbase = lb * D
        for c in range(2):
            for par in range(2):
                plsc.store_scatter(
                    pooled_v,
                    [jnp.full((16,), pbase + 32 * c + par, jnp.int32) + col2],
                    pooled[2 * c + par])

        pltpu.async_copy(cur["oacc"],
                         out_hbm.at[pl.ds(b * L1 * D, L1 * D)],
                         cur["sem_out"])

        @pl.when(lb + 2 < B_PER_W)
        def _ids_next():
            pltpu.async_copy(ids_hbm.at[pl.ds((b + 2) * EPB, EPB)],
                             cur["idx"].at[pl.ds(0, EPB)], cur["sem_ids"])

    @pl.loop(0, B_PER_W // 2)
    def _pair_loop(t):
        stage(2 * t, bufs[0], bufs[1])
        stage(2 * t + 1, bufs[1], bufs[0])

    for par, bf in enumerate(bufs):
        pltpu.make_async_copy(
            bf["oacc"],
            out_hbm.at[pl.ds((b0 + B_PER_W - 2 + par) * L1 * D, L1 * D)],
            bf["sem_out"]).wait()
    pltpu.sync_copy(pooled_v,
                    pooled_hbm.at[pl.ds(wid * B_PER_W * D, B_PER_W * D)])


def _head_body(pooled_ref, w_ref, b_ref, out_ref):
    logits = jnp.dot(pooled_ref[...], w_ref[...].T,
                     preferred_element_type=jnp.float32) + b_ref[0, 0]
    out_ref[...] = jax.nn.sigmoid(logits)


_HEAD_PAD = 8


def kernel(medication_ids, dose, units, med_table, units_table, W, b):
    ids_flat = medication_ids.astype(jnp.int32).reshape(B * L1 * L2)
    units_flat = units.astype(jnp.int32).reshape(B * L1 * L2)
    dose_flat = dose.reshape(B * L1 * L2)
    med_bf16 = med_table.astype(jnp.bfloat16)
    utab_de = (units_table.reshape(V_UNITS + 1, 2, 16, 2)
               .transpose(0, 1, 3, 2).reshape((V_UNITS + 1) * D))

    sc = pl.kernel(
        _sc_body,
        out_type=[
            jax.ShapeDtypeStruct((B * L1 * D,), jnp.float32),
            jax.ShapeDtypeStruct((B * D,), jnp.float32),
        ],
        mesh=plsc.VectorSubcoreMesh(core_axis_name="c", subcore_axis_name="s"),
        compiler_params=pltpu.CompilerParams(use_tc_tiling_on_sc=False,
                                             needs_layout_passes=False),
        scratch_types=[
            pltpu.VMEM((EPB_PAD,), jnp.int32),
            pltpu.VMEM((EPB_PAD,), jnp.int32),
            pltpu.VMEM((EPB_PAD,), jnp.int32),
            pltpu.VMEM((EPB_PAD,), jnp.int32),
            pltpu.VMEM((EPB_PAD,), jnp.float32),
            pltpu.VMEM((EPB_PAD,), jnp.float32),
            pltpu.VMEM((L1 * L2P,), jnp.int32),
            pltpu.VMEM((L1 * L2P,), jnp.int32),
            pltpu.VMEM((L1 * L2P,), jnp.float32),
            pltpu.VMEM((L1 * L2P,), jnp.float32),
            pltpu.VMEM((EPB_PAD, D), jnp.bfloat16),
            pltpu.VMEM((EPB_PAD, D), jnp.bfloat16),
            pltpu.VMEM((L1 * D,), jnp.float32),
            pltpu.VMEM((L1 * D,), jnp.float32),
            pltpu.VMEM(((V_UNITS + 1) * D,), jnp.float32),
            pltpu.VMEM((B_PER_W * D,), jnp.float32),
            pltpu.SemaphoreType.DMA,
            pltpu.SemaphoreType.DMA,
            pltpu.SemaphoreType.DMA,
            pltpu.SemaphoreType.DMA,
            pltpu.SemaphoreType.DMA,
            pltpu.SemaphoreType.DMA,
            pltpu.SemaphoreType.DMA,
            pltpu.SemaphoreType.DMA,
        ],
    )
    out_flat, pooled_flat = sc(ids_flat, units_flat, dose_flat,
                               med_bf16, utab_de)

    pooled = pooled_flat.reshape(B, D)
    w_pad = jnp.pad(W, ((0, _HEAD_PAD - 1), (0, 0)))
    outcome = pl.pallas_call(
        _head_body,
        out_shape=jax.ShapeDtypeStruct((B, _HEAD_PAD), jnp.float32),
    )(pooled, w_pad, b.reshape(1, 1))

    return out_flat.reshape(B, L1, D), outcome[:, :1]


# speedup vs baseline: 14.3196x; 1.0004x over previous
import functools

import jax
import jax.numpy as jnp
from jax import lax
from jax.experimental import pallas as pl
from jax.experimental.pallas import tpu as pltpu
from jax.experimental.pallas import tpu_sc as plsc

B, L1, L2, D = 1024, 50, 20, 64
V_MED, V_UNITS = 100000, 100
NC, NS = 2, 16
NW = NC * NS
B_PER_W = B // NW
EPB = L1 * L2
EPB_PAD = 1024
GCHUNK = 128
NG = EPB_PAD // GCHUNK
L2P = 32


def _sc_body(ids_hbm, units_hbm, dose_hbm, med_tab_hbm, utab_hbm,
             out_hbm, pooled_hbm, *refs):
    (idx0, idx1, stag_u0, stag_u1, stag_d0, stag_d1,
     rows0, rows1, oacc0, oacc1,
     utab_v, pooled_v,
     sem_ids0, sem_ids1, sem_inp0, sem_inp1,
     sem_g0, sem_g1, sem_out0, sem_out1) = refs
    bufs = [
        dict(idx=idx0, stag_u=stag_u0, stag_d=stag_d0,
             rows=rows0, oacc=oacc0, sem_ids=sem_ids0,
             sem_inp=sem_inp0, sem_g=sem_g0, sem_out=sem_out0),
        dict(idx=idx1, stag_u=stag_u1, stag_d=stag_d1,
             rows=rows1, oacc=oacc1, sem_ids=sem_ids1,
             sem_inp=sem_inp1, sem_g=sem_g1, sem_out=sem_out1),
    ]
    wid = lax.axis_index("s") * NC + lax.axis_index("c")
    b0 = wid * B_PER_W
    col16 = lax.iota(jnp.int32, 16)
    col2 = col16 * 2
    zeros16 = jnp.zeros((16,), jnp.float32)
    himask = jnp.full((16,), -65536, jnp.int32)
    cp = [[col16 + (32 * c + 16 * p) for p in range(2)] for c in range(2)]

    pltpu.sync_copy(utab_hbm, utab_v)
    for bf in bufs:
        bf["idx"][pl.ds(EPB, 16)] = jnp.zeros((16,), jnp.int32)
        bf["idx"][pl.ds(EPB + 8, 16)] = jnp.zeros((16,), jnp.int32)
    d_ids0 = pltpu.async_copy(ids_hbm.at[pl.ds(b0 * EPB, EPB)],
                              idx0.at[pl.ds(0, EPB)], sem_ids0)
    pltpu.async_copy(ids_hbm.at[pl.ds((b0 + 1) * EPB, EPB)],
                     idx1.at[pl.ds(0, EPB)], sem_ids1)
    d_ids0.wait()
    for j in range(NG):
        pltpu.async_copy(
            med_tab_hbm.at[idx0.at[pl.ds(j * GCHUNK, GCHUNK)]],
            rows0.at[pl.ds(j * GCHUNK, GCHUNK)], sem_g0)
    pltpu.async_copy(units_hbm.at[pl.ds(b0 * EPB, EPB)],
                     stag_u0.at[pl.ds(0, EPB)], sem_inp0)
    pltpu.async_copy(dose_hbm.at[pl.ds(b0 * EPB, EPB)],
                     stag_d0.at[pl.ds(0, EPB)], sem_inp0)
    pltpu.async_copy(oacc0, out_hbm.at[pl.ds(b0 * L1 * D, L1 * D)],
                     sem_out0)
    pltpu.async_copy(oacc1, out_hbm.at[pl.ds((b0 + 1) * L1 * D, L1 * D)],
                     sem_out1)

    def stage(lb, cur, nxt):
        b = b0 + lb
        @pl.when(lb + 1 < B_PER_W)
        def _prefetch():
            bn = b + 1
            pltpu.make_async_copy(
                ids_hbm.at[pl.ds(bn * EPB, EPB)],
                nxt["idx"].at[pl.ds(0, EPB)], nxt["sem_ids"]).wait()
            for j in range(NG):
                pltpu.async_copy(
                    med_tab_hbm.at[nxt["idx"].at[pl.ds(j * GCHUNK, GCHUNK)]],
                    nxt["rows"].at[pl.ds(j * GCHUNK, GCHUNK)], nxt["sem_g"])
            pltpu.async_copy(units_hbm.at[pl.ds(bn * EPB, EPB)],
                             nxt["stag_u"].at[pl.ds(0, EPB)], nxt["sem_inp"])
            pltpu.async_copy(dose_hbm.at[pl.ds(bn * EPB, EPB)],
                             nxt["stag_d"].at[pl.ds(0, EPB)], nxt["sem_inp"])

        pltpu.make_async_copy(units_hbm.at[pl.ds(b * EPB, EPB)],
                              cur["stag_u"].at[pl.ds(0, EPB)],
                              cur["sem_inp"]).wait()
        pltpu.make_async_copy(dose_hbm.at[pl.ds(b * EPB, EPB)],
                              cur["stag_d"].at[pl.ds(0, EPB)],
                              cur["sem_inp"]).wait()

        for j in range(NG):
            pltpu.make_async_copy(
                med_tab_hbm.at[cur["idx"].at[pl.ds(j * GCHUNK, GCHUNK)]],
                cur["rows"].at[pl.ds(j * GCHUNK, GCHUNK)], cur["sem_g"]).wait()
        pltpu.make_async_copy(
            cur["oacc"], out_hbm.at[pl.ds(b * L1 * D, L1 * D)],
            cur["sem_out"]).wait()

        def l1_body(l1, pooled):
            acc = [zeros16, zeros16, zeros16, zeros16]
            for l2 in range(L2):
                k = l1 * L2 + l2
                eli = jnp.full((16,), k, jnp.int32)
                dvec = plsc.load_gather(cur["stag_d"], [eli])
                ubase = lax.shift_left(
                    plsc.load_gather(cur["stag_u"], [eli]), 6)
                for c in range(2):
                    v = plsc.bitcast(cur["rows"][k, pl.ds(32 * c, 32)],
                                     jnp.int32)
                    m_e = plsc.bitcast(lax.shift_left(v, 16), jnp.float32)
                    m_o = plsc.bitcast(v & himask, jnp.float32)
                    u_e = plsc.load_gather(utab_v, [ubase + cp[c][0]])
                    u_o = plsc.load_gather(utab_v, [ubase + cp[c][1]])
                    acc[2 * c] = acc[2 * c] + m_e * u_e * dvec
                    acc[2 * c + 1] = acc[2 * c + 1] + m_o * u_o * dvec
            base = l1 * D
            for c in range(2):
                for par in range(2):
                    plsc.store_scatter(
                        cur["oacc"],
                        [jnp.full((16,), base + 32 * c + par, jnp.int32)
                         + col2],
                        acc[2 * c + par])
            return [pooled[i] + acc[i] for i in range(4)]

        pooled = lax.fori_loop(0, L1, l1_body,
                               [zeros16, zeros16, zeros16, zeros16])
        pbase = lb * D
        for c in range(2):
            for par in range(2):
                plsc.store_scatter(
                    pooled_v,
                    [jnp.full((16,), pbase + 32 * c + par, jnp.int32) + col2],
                    pooled[2 * c + par])

        pltpu.async_copy(cur["oacc"],
                         out_hbm.at[pl.ds(b * L1 * D, L1 * D)],
                         cur["sem_out"])

        @pl.when(lb + 2 < B_PER_W)
        def _ids_next():
            pltpu.async_copy(ids_hbm.at[pl.ds((b + 2) * EPB, EPB)],
                             cur["idx"].at[pl.ds(0, EPB)], cur["sem_ids"])

    @pl.loop(0, B_PER_W // 2)
    def _pair_loop(t):
        stage(2 * t, bufs[0], bufs[1])
        stage(2 * t + 1, bufs[1], bufs[0])

    for par, bf in enumerate(bufs):
        pltpu.make_async_copy(
            bf["oacc"],
            out_hbm.at[pl.ds((b0 + B_PER_W - 2 + par) * L1 * D, L1 * D)],
            bf["sem_out"]).wait()
    pltpu.sync_copy(pooled_v,
                    pooled_hbm.at[pl.ds(wid * B_PER_W * D, B_PER_W * D)])


def _head_body(pooled_ref, w_ref, b_ref, out_ref):
    logits = jnp.dot(pooled_ref[...], w_ref[...].T,
                     preferred_element_type=jnp.float32) + b_ref[0, 0]
    out_ref[...] = jax.nn.sigmoid(logits)


_HEAD_PAD = 8


def kernel(medication_ids, dose, units, med_table, units_table, W, b):
    ids_flat = medication_ids.astype(jnp.int32).reshape(B * L1 * L2)
    units_flat = units.astype(jnp.int32).reshape(B * L1 * L2)
    dose_flat = dose.reshape(B * L1 * L2)
    med_bf16 = med_table.astype(jnp.bfloat16)
    utab_de = (units_table.reshape(V_UNITS + 1, 2, 16, 2)
               .transpose(0, 1, 3, 2).reshape((V_UNITS + 1) * D))

    sc = pl.kernel(
        _sc_body,
        out_type=[
            jax.ShapeDtypeStruct((B * L1 * D,), jnp.float32),
            jax.ShapeDtypeStruct((B * D,), jnp.float32),
        ],
        mesh=plsc.VectorSubcoreMesh(core_axis_name="c", subcore_axis_name="s"),
        compiler_params=pltpu.CompilerParams(use_tc_tiling_on_sc=False,
                                             needs_layout_passes=False),
        scratch_types=[
            pltpu.VMEM((EPB_PAD,), jnp.int32),
            pltpu.VMEM((EPB_PAD,), jnp.int32),
            pltpu.VMEM((EPB_PAD,), jnp.int32),
            pltpu.VMEM((EPB_PAD,), jnp.int32),
            pltpu.VMEM((EPB_PAD,), jnp.float32),
            pltpu.VMEM((EPB_PAD,), jnp.float32),
            pltpu.VMEM((EPB_PAD, D), jnp.bfloat16),
            pltpu.VMEM((EPB_PAD, D), jnp.bfloat16),
            pltpu.VMEM((L1 * D,), jnp.float32),
            pltpu.VMEM((L1 * D,), jnp.float32),
            pltpu.VMEM(((V_UNITS + 1) * D,), jnp.float32),
            pltpu.VMEM((B_PER_W * D,), jnp.float32),
            pltpu.SemaphoreType.DMA,
            pltpu.SemaphoreType.DMA,
            pltpu.SemaphoreType.DMA,
            pltpu.SemaphoreType.DMA,
            pltpu.SemaphoreType.DMA,
            pltpu.SemaphoreType.DMA,
            pltpu.SemaphoreType.DMA,
            pltpu.SemaphoreType.DMA,
        ],
    )
    out_flat, pooled_flat = sc(ids_flat, units_flat, dose_flat,
                               med_bf16, utab_de)

    pooled = pooled_flat.reshape(B, D)
    w_pad = jnp.pad(W, ((0, _HEAD_PAD - 1), (0, 0)))
    outcome = pl.pallas_call(
        _head_body,
        out_shape=jax.ShapeDtypeStruct((B, _HEAD_PAD), jnp.float32),
    )(pooled, w_pad, b.reshape(1, 1))

    return out_flat.reshape(B, L1, D), outcome[:, :1]


# speedup vs baseline: 15.2997x; 1.0684x over previous
import functools

import jax
import jax.numpy as jnp
from jax import lax
from jax.experimental import pallas as pl
from jax.experimental.pallas import tpu as pltpu
from jax.experimental.pallas import tpu_sc as plsc

B, L1, L2, D = 1024, 50, 20, 64
V_MED, V_UNITS = 100000, 100
NC, NS = 2, 16
NW = NC * NS
B_PER_W = B // NW
EPB = L1 * L2
EPB_PAD = 1024
GCHUNK = 128
NG = EPB_PAD // GCHUNK
L2P = 32


def _sc_body(pk_hbm, dose_hbm, med_tab_hbm, utab_hbm,
             out_hbm, pooled_hbm, *refs):
    (pk0, pk1, idx0, idx1, ub0, ub1, stag_d0, stag_d1,
     rows0, rows1, oacc0, oacc1,
     utab_v, pooled_v,
     sem_ids0, sem_ids1, sem_inp0, sem_inp1,
     sem_g0, sem_g1, sem_out0, sem_out1) = refs
    bufs = [
        dict(pk=pk0, idx=idx0, ub=ub0, stag_d=stag_d0,
             rows=rows0, oacc=oacc0, sem_ids=sem_ids0,
             sem_inp=sem_inp0, sem_g=sem_g0, sem_out=sem_out0),
        dict(pk=pk1, idx=idx1, ub=ub1, stag_d=stag_d1,
             rows=rows1, oacc=oacc1, sem_ids=sem_ids1,
             sem_inp=sem_inp1, sem_g=sem_g1, sem_out=sem_out1),
    ]
    wid = lax.axis_index("s") * NC + lax.axis_index("c")
    b0 = wid * B_PER_W
    col16 = lax.iota(jnp.int32, 16)
    col2 = col16 * 2
    zeros16 = jnp.zeros((16,), jnp.float32)
    zeros16i = jnp.zeros((16,), jnp.int32)
    himask = jnp.full((16,), -65536, jnp.int32)
    idmask = jnp.full((16,), 0x1FFFF, jnp.int32)
    ubmask = jnp.full((16,), 0x7F << 6, jnp.int32)
    cp = [[col16 + (32 * c + 16 * p) for p in range(2)] for c in range(2)]

    def unpack(bf):
        @pl.loop(0, EPB_PAD // 16)
        def _u(w):
            p = bf["pk"][pl.ds(w * 16, 16)]
            bf["idx"][pl.ds(w * 16, 16)] = p & idmask
            bf["ub"][pl.ds(w * 16, 16)] = (
                lax.shift_right_logical(p, 11) & ubmask)
        bf["idx"][pl.ds(EPB, 16)] = zeros16i
        bf["idx"][pl.ds(EPB + 8, 16)] = zeros16i

    pltpu.sync_copy(utab_hbm, utab_v)
    d_ids0 = pltpu.async_copy(pk_hbm.at[pl.ds(b0 * EPB, EPB)],
                              pk0.at[pl.ds(0, EPB)], sem_ids0)
    pltpu.async_copy(pk_hbm.at[pl.ds((b0 + 1) * EPB, EPB)],
                     pk1.at[pl.ds(0, EPB)], sem_ids1)
    d_ids0.wait()
    unpack(bufs[0])
    for j in range(NG):
        pltpu.async_copy(
            med_tab_hbm.at[idx0.at[pl.ds(j * GCHUNK, GCHUNK)]],
            rows0.at[pl.ds(j * GCHUNK, GCHUNK)], sem_g0)
    pltpu.async_copy(dose_hbm.at[pl.ds(b0 * EPB, EPB)],
                     stag_d0.at[pl.ds(0, EPB)], sem_inp0)
    pltpu.async_copy(oacc0, out_hbm.at[pl.ds(b0 * L1 * D, L1 * D)],
                     sem_out0)
    pltpu.async_copy(oacc1, out_hbm.at[pl.ds((b0 + 1) * L1 * D, L1 * D)],
                     sem_out1)

    def stage(lb, cur, nxt):
        b = b0 + lb
        @pl.when(lb + 1 < B_PER_W)
        def _prefetch():
            bn = b + 1
            pltpu.make_async_copy(
                pk_hbm.at[pl.ds(bn * EPB, EPB)],
                nxt["pk"].at[pl.ds(0, EPB)], nxt["sem_ids"]).wait()
            unpack(nxt)
            for j in range(NG):
                pltpu.async_copy(
                    med_tab_hbm.at[nxt["idx"].at[pl.ds(j * GCHUNK, GCHUNK)]],
                    nxt["rows"].at[pl.ds(j * GCHUNK, GCHUNK)], nxt["sem_g"])
            pltpu.async_copy(dose_hbm.at[pl.ds(bn * EPB, EPB)],
                             nxt["stag_d"].at[pl.ds(0, EPB)], nxt["sem_inp"])

        pltpu.make_async_copy(dose_hbm.at[pl.ds(b * EPB, EPB)],
                              cur["stag_d"].at[pl.ds(0, EPB)],
                              cur["sem_inp"]).wait()

        for j in range(NG):
            pltpu.make_async_copy(
                med_tab_hbm.at[cur["idx"].at[pl.ds(j * GCHUNK, GCHUNK)]],
                cur["rows"].at[pl.ds(j * GCHUNK, GCHUNK)], cur["sem_g"]).wait()
        pltpu.make_async_copy(
            cur["oacc"], out_hbm.at[pl.ds(b * L1 * D, L1 * D)],
            cur["sem_out"]).wait()

        def l1_body(l1, pooled):
            acc = [zeros16, zeros16, zeros16, zeros16]
            for l2 in range(L2):
                k = l1 * L2 + l2
                eli = jnp.full((16,), k, jnp.int32)
                dvec = plsc.load_gather(cur["stag_d"], [eli])
                ubase = plsc.load_gather(cur["ub"], [eli])
                for c in range(2):
                    v = plsc.bitcast(cur["rows"][k, pl.ds(32 * c, 32)],
                                     jnp.int32)
                    m_e = plsc.bitcast(lax.shift_left(v, 16), jnp.float32)
                    m_o = plsc.bitcast(v & himask, jnp.float32)
                    u_e = plsc.load_gather(utab_v, [ubase + cp[c][0]])
                    u_o = plsc.load_gather(utab_v, [ubase + cp[c][1]])
                    acc[2 * c] = acc[2 * c] + m_e * u_e * dvec
                    acc[2 * c + 1] = acc[2 * c + 1] + m_o * u_o * dvec
            base = l1 * D
            for c in range(2):
                for par in range(2):
                    plsc.store_scatter(
                        cur["oacc"],
                        [jnp.full((16,), base + 32 * c + par, jnp.int32)
                         + col2],
                        acc[2 * c + par])
            return [pooled[i] + acc[i] for i in range(4)]

        pooled = lax.fori_loop(0, L1, l1_body,
                               [zeros16, zeros16, zeros16, zeros16])
        pbase = lb * D
        for c in range(2):
            for par in range(2):
                plsc.store_scatter(
                    pooled_v,
                    [jnp.full((16,), pbase + 32 * c + par, jnp.int32) + col2],
                    pooled[2 * c + par])

        pltpu.async_copy(cur["oacc"],
                         out_hbm.at[pl.ds(b * L1 * D, L1 * D)],
                         cur["sem_out"])

        @pl.when(lb + 2 < B_PER_W)
        def _ids_next():
            pltpu.async_copy(pk_hbm.at[pl.ds((b + 2) * EPB, EPB)],
                             cur["pk"].at[pl.ds(0, EPB)], cur["sem_ids"])

    @pl.loop(0, B_PER_W // 2)
    def _pair_loop(t):
        stage(2 * t, bufs[0], bufs[1])
        stage(2 * t + 1, bufs[1], bufs[0])

    for par, bf in enumerate(bufs):
        pltpu.make_async_copy(
            bf["oacc"],
            out_hbm.at[pl.ds((b0 + B_PER_W - 2 + par) * L1 * D, L1 * D)],
            bf["sem_out"]).wait()
    pltpu.sync_copy(pooled_v,
                    pooled_hbm.at[pl.ds(wid * B_PER_W * D, B_PER_W * D)])


def _head_body(pooled_ref, w_ref, b_ref, out_ref):
    logits = jnp.dot(pooled_ref[...], w_ref[...].T,
                     preferred_element_type=jnp.float32) + b_ref[0, 0]
    out_ref[...] = jax.nn.sigmoid(logits)


_HEAD_PAD = 8


def kernel(medication_ids, dose, units, med_table, units_table, W, b):
    packed = (medication_ids.astype(jnp.int32)
              | (units.astype(jnp.int32) << 17))
    pk_flat = packed.reshape(B * L1 * L2)
    dose_flat = dose.reshape(B * L1 * L2)
    med_bf16 = med_table.astype(jnp.bfloat16)
    utab_de = (units_table.reshape(V_UNITS + 1, 2, 16, 2)
               .transpose(0, 1, 3, 2).reshape((V_UNITS + 1) * D))

    sc = pl.kernel(
        _sc_body,
        out_type=[
            jax.ShapeDtypeStruct((B * L1 * D,), jnp.float32),
            jax.ShapeDtypeStruct((B * D,), jnp.float32),
        ],
        mesh=plsc.VectorSubcoreMesh(core_axis_name="c", subcore_axis_name="s"),
        compiler_params=pltpu.CompilerParams(use_tc_tiling_on_sc=False,
                                             needs_layout_passes=False),
        scratch_types=[
            pltpu.VMEM((EPB_PAD,), jnp.int32),
            pltpu.VMEM((EPB_PAD,), jnp.int32),
            pltpu.VMEM((EPB_PAD,), jnp.int32),
            pltpu.VMEM((EPB_PAD,), jnp.int32),
            pltpu.VMEM((EPB_PAD,), jnp.int32),
            pltpu.VMEM((EPB_PAD,), jnp.int32),
            pltpu.VMEM((EPB_PAD,), jnp.float32),
            pltpu.VMEM((EPB_PAD,), jnp.float32),
            pltpu.VMEM((EPB_PAD, D), jnp.bfloat16),
            pltpu.VMEM((EPB_PAD, D), jnp.bfloat16),
            pltpu.VMEM((L1 * D,), jnp.float32),
            pltpu.VMEM((L1 * D,), jnp.float32),
            pltpu.VMEM(((V_UNITS + 1) * D,), jnp.float32),
            pltpu.VMEM((B_PER_W * D,), jnp.float32),
            pltpu.SemaphoreType.DMA,
            pltpu.SemaphoreType.DMA,
            pltpu.SemaphoreType.DMA,
            pltpu.SemaphoreType.DMA,
            pltpu.SemaphoreType.DMA,
            pltpu.SemaphoreType.DMA,
            pltpu.SemaphoreType.DMA,
            pltpu.SemaphoreType.DMA,
        ],
    )
    out_flat, pooled_flat = sc(pk_flat, dose_flat, med_bf16, utab_de)

    pooled = pooled_flat.reshape(B, D)
    w_pad = jnp.pad(W, ((0, _HEAD_PAD - 1), (0, 0)))
    outcome = pl.pallas_call(
        _head_body,
        out_shape=jax.ShapeDtypeStruct((B, _HEAD_PAD), jnp.float32),
    )(pooled, w_pad, b.reshape(1, 1))

    return out_flat.reshape(B, L1, D), outcome[:, :1]
